# baseline (device time: 67584 ns/iter reference)
import jax
import jax.numpy as jnp
from jax import lax
from jax.experimental import pallas as pl
from jax.experimental.pallas import tpu as pltpu

B, S, HL, D = 2, 1024, 16, 64
K = HL * D
N = 2048
S_HALF = S // 2


def kernel(O, Wo):
    O2 = O.reshape(B, S, K)

    def body(o_ref, w_ref, out_ref, send_buf, recv_buf, send_sem, recv_sem):
        my_x = lax.axis_index("x")
        my_y = lax.axis_index("y")
        peer_y = 1 - my_y

        barrier_sem = pltpu.get_barrier_semaphore()
        pl.semaphore_signal(
            barrier_sem, inc=1,
            device_id=(my_x, peer_y), device_id_type=pl.DeviceIdType.MESH,
        )
        pl.semaphore_wait(barrier_sem, 1)

        w = w_ref[...].astype(jnp.bfloat16)

        for b in range(B):
            o_blk = o_ref[b, pl.ds(peer_y * S_HALF, S_HALF), :].astype(jnp.bfloat16)
            send_buf[b, :, :] = jnp.dot(
                o_blk, w, preferred_element_type=jnp.float32
            ).astype(jnp.bfloat16)

        rdma = pltpu.make_async_remote_copy(
            src_ref=send_buf,
            dst_ref=recv_buf,
            send_sem=send_sem,
            recv_sem=recv_sem,
            device_id=(my_x, peer_y),
            device_id_type=pl.DeviceIdType.MESH,
        )
        rdma.start()

        for b in range(B):
            o_blk = o_ref[b, pl.ds(my_y * S_HALF, S_HALF), :].astype(jnp.bfloat16)
            out_ref[b, :, :] = jnp.dot(
                o_blk, w, preferred_element_type=jnp.float32
            )

        rdma.wait()
        for b in range(B):
            out_ref[b, :, :] = out_ref[b, :, :] + recv_buf[b, :, :].astype(jnp.float32)

    return pl.pallas_call(
        body,
        out_shape=jax.ShapeDtypeStruct((B, S_HALF, N), jnp.float32),
        in_specs=[
            pl.BlockSpec(memory_space=pltpu.VMEM),
            pl.BlockSpec(memory_space=pltpu.VMEM),
        ],
        out_specs=pl.BlockSpec(memory_space=pltpu.VMEM),
        scratch_shapes=[
            pltpu.VMEM((B, S_HALF, N), jnp.bfloat16),
            pltpu.VMEM((B, S_HALF, N), jnp.bfloat16),
            pltpu.SemaphoreType.DMA,
            pltpu.SemaphoreType.DMA,
        ],
        compiler_params=pltpu.CompilerParams(collective_id=0),
    )(O2, Wo)


# device time: 48126 ns/iter; 1.4043x vs baseline; 1.4043x over previous
import jax
import jax.numpy as jnp
from jax import lax
from jax.experimental import pallas as pl
from jax.experimental.pallas import tpu as pltpu

B, S, HL, D = 2, 1024, 16, 64
K = HL * D
N = 2048
NH = N // 2
S_HALF = S // 2
RCH = 128
NC_PER_B = S_HALF // RCH
NCH = B * NC_PER_B


def kernel(O, Wo):
    O2 = O.reshape(B, S, K)

    def body(o_ref, w_ref, out_ref,
             p1_send, p1_recv, p2_send, p2_recv,
             s1_sems, r1_sems, s2_sems, r2_sems):
        my_x = lax.axis_index("x")
        my_y = lax.axis_index("y")
        peer_y = 1 - my_y
        peer_x = 1 - my_x

        barrier_sem = pltpu.get_barrier_semaphore()
        pl.semaphore_signal(
            barrier_sem, inc=1,
            device_id=(my_x, peer_y), device_id_type=pl.DeviceIdType.MESH,
        )
        pl.semaphore_signal(
            barrier_sem, inc=1,
            device_id=(peer_x, my_y), device_id_type=pl.DeviceIdType.MESH,
        )
        pl.semaphore_wait(barrier_sem, 2)

        w_my = w_ref[:, pl.ds(my_x * NH, NH)].astype(jnp.bfloat16)

        rdma1 = []
        for c in range(NCH):
            b, sh = divmod(c, NC_PER_B)
            row0 = peer_y * S_HALF + sh * RCH
            o_blk = o_ref[b, pl.ds(row0, RCH), :].astype(jnp.bfloat16)
            p1_send[c, :, :] = jnp.dot(
                o_blk, w_my, preferred_element_type=jnp.float32
            ).astype(jnp.bfloat16)
            r = pltpu.make_async_remote_copy(
                src_ref=p1_send.at[c],
                dst_ref=p1_recv.at[c],
                send_sem=s1_sems.at[c],
                recv_sem=r1_sems.at[c],
                device_id=(my_x, peer_y),
                device_id_type=pl.DeviceIdType.MESH,
            )
            r.start()
            rdma1.append(r)

        for c in range(NCH):
            b, sh = divmod(c, NC_PER_B)
            row0 = my_y * S_HALF + sh * RCH
            o_blk = o_ref[b, pl.ds(row0, RCH), :].astype(jnp.bfloat16)
            p2_send[c, :, :] = jnp.dot(
                o_blk, w_my, preferred_element_type=jnp.float32
            ).astype(jnp.bfloat16)

        rdma2 = []
        for c in range(NCH):
            b, sh = divmod(c, NC_PER_B)
            rdma1[c].wait_recv()
            p2_send[c, :, :] = p2_send[c, :, :] + p1_recv[c, :, :]
            r = pltpu.make_async_remote_copy(
                src_ref=p2_send.at[c],
                dst_ref=p2_recv.at[c],
                send_sem=s2_sems.at[c],
                recv_sem=r2_sems.at[c],
                device_id=(peer_x, my_y),
                device_id_type=pl.DeviceIdType.MESH,
            )
            r.start()
            rdma2.append(r)
            out_ref[b, pl.ds(sh * RCH, RCH), pl.ds(my_x * NH, NH)] = (
                p2_send[c, :, :].astype(jnp.float32)
            )

        for c in range(NCH):
            b, sh = divmod(c, NC_PER_B)
            rdma2[c].wait_recv()
            out_ref[b, pl.ds(sh * RCH, RCH), pl.ds(peer_x * NH, NH)] = (
                p2_recv[c, :, :].astype(jnp.float32)
            )

        for c in range(NCH):
            rdma1[c].wait_send()
            rdma2[c].wait_send()

    return pl.pallas_call(
        body,
        out_shape=jax.ShapeDtypeStruct((B, S_HALF, N), jnp.float32),
        in_specs=[
            pl.BlockSpec(memory_space=pltpu.VMEM),
            pl.BlockSpec(memory_space=pltpu.VMEM),
        ],
        out_specs=pl.BlockSpec(memory_space=pltpu.VMEM),
        scratch_shapes=[
            pltpu.VMEM((NCH, RCH, NH), jnp.bfloat16),
            pltpu.VMEM((NCH, RCH, NH), jnp.bfloat16),
            pltpu.VMEM((NCH, RCH, NH), jnp.bfloat16),
            pltpu.VMEM((NCH, RCH, NH), jnp.bfloat16),
            pltpu.SemaphoreType.DMA((NCH,)),
            pltpu.SemaphoreType.DMA((NCH,)),
            pltpu.SemaphoreType.DMA((NCH,)),
            pltpu.SemaphoreType.DMA((NCH,)),
        ],
        compiler_params=pltpu.CompilerParams(collective_id=0),
    )(O2, Wo)


# device time: 47855 ns/iter; 1.4123x vs baseline; 1.0057x over previous
import jax
import jax.numpy as jnp
from jax import lax
from jax.experimental import pallas as pl
from jax.experimental.pallas import tpu as pltpu

B, S, HL, D = 2, 1024, 16, 64
K = HL * D
N = 2048
NH = N // 2
S_HALF = S // 2
RCH = 128
NC_PER_B = S_HALF // RCH
NCH = B * NC_PER_B


def kernel(O, Wo):
    O2 = O.reshape(B, S, K)

    def body(o_ref, w_ref, out_ref,
             p1_send, p1_recv, p2_send, p2_recv,
             s1_sems, r1_sems, s2_sems, r2_sems):
        my_x = lax.axis_index("x")
        my_y = lax.axis_index("y")
        peer_y = 1 - my_y
        peer_x = 1 - my_x

        barrier_sem = pltpu.get_barrier_semaphore()
        pl.semaphore_signal(
            barrier_sem, inc=1,
            device_id=(my_x, peer_y), device_id_type=pl.DeviceIdType.MESH,
        )
        pl.semaphore_signal(
            barrier_sem, inc=1,
            device_id=(peer_x, my_y), device_id_type=pl.DeviceIdType.MESH,
        )
        pl.semaphore_wait(barrier_sem, 2)

        w_my = w_ref[:, pl.ds(my_x * NH, NH)].astype(jnp.bfloat16)

        rdma1 = []
        for c in range(NCH):
            b, sh = divmod(c, NC_PER_B)
            row0 = peer_y * S_HALF + sh * RCH
            o_blk = o_ref[b, pl.ds(row0, RCH), :].astype(jnp.bfloat16)
            p1_send[c, :, :] = jnp.dot(
                o_blk, w_my, preferred_element_type=jnp.float32
            ).astype(jnp.bfloat16)
            r = pltpu.make_async_remote_copy(
                src_ref=p1_send.at[c],
                dst_ref=p1_recv.at[c],
                send_sem=s1_sems.at[c],
                recv_sem=r1_sems.at[c],
                device_id=(my_x, peer_y),
                device_id_type=pl.DeviceIdType.MESH,
            )
            r.start()
            rdma1.append(r)
            row0m = my_y * S_HALF + sh * RCH
            o_blkm = o_ref[b, pl.ds(row0m, RCH), :].astype(jnp.bfloat16)
            p2_send[c, :, :] = jnp.dot(
                o_blkm, w_my, preferred_element_type=jnp.float32
            ).astype(jnp.bfloat16)

        rdma2 = []
        for c in range(NCH):
            b, sh = divmod(c, NC_PER_B)
            rdma1[c].wait_recv()
            p2_send[c, :, :] = p2_send[c, :, :] + p1_recv[c, :, :]
            r = pltpu.make_async_remote_copy(
                src_ref=p2_send.at[c],
                dst_ref=p2_recv.at[c],
                send_sem=s2_sems.at[c],
                recv_sem=r2_sems.at[c],
                device_id=(peer_x, my_y),
                device_id_type=pl.DeviceIdType.MESH,
            )
            r.start()
            rdma2.append(r)
            out_ref[b, pl.ds(sh * RCH, RCH), pl.ds(my_x * NH, NH)] = (
                p2_send[c, :, :].astype(jnp.float32)
            )

        for c in range(NCH):
            b, sh = divmod(c, NC_PER_B)
            rdma2[c].wait_recv()
            out_ref[b, pl.ds(sh * RCH, RCH), pl.ds(peer_x * NH, NH)] = (
                p2_recv[c, :, :].astype(jnp.float32)
            )

        for c in range(NCH):
            rdma1[c].wait_send()
            rdma2[c].wait_send()

    return pl.pallas_call(
        body,
        out_shape=jax.ShapeDtypeStruct((B, S_HALF, N), jnp.float32),
        in_specs=[
            pl.BlockSpec(memory_space=pltpu.VMEM),
            pl.BlockSpec(memory_space=pltpu.VMEM),
        ],
        out_specs=pl.BlockSpec(memory_space=pltpu.VMEM),
        scratch_shapes=[
            pltpu.VMEM((NCH, RCH, NH), jnp.bfloat16),
            pltpu.VMEM((NCH, RCH, NH), jnp.bfloat16),
            pltpu.VMEM((NCH, RCH, NH), jnp.bfloat16),
            pltpu.VMEM((NCH, RCH, NH), jnp.bfloat16),
            pltpu.SemaphoreType.DMA((NCH,)),
            pltpu.SemaphoreType.DMA((NCH,)),
            pltpu.SemaphoreType.DMA((NCH,)),
            pltpu.SemaphoreType.DMA((NCH,)),
        ],
        compiler_params=pltpu.CompilerParams(collective_id=0),
    )(O2, Wo)
